# baseline (speedup 1.0000x reference)
import jax
import jax.numpy as jnp
from jax import lax
from jax.experimental import pallas as pl
from jax.experimental.pallas import tpu as pltpu
from jax.experimental.pallas import tpu_sc as plsc

_D = 64
_B = 4096 * 200
_NW = 32
_PER_W = _B // _NW
_CHUNK = 128
_NCHUNK = _PER_W // _CHUNK
_SCALE = 8.0

_LANES = 16


def _body(idx_hbm, table_hbm, out_hbm, idx_v, buf_v, gsem):
    c = lax.axis_index("c")
    s = lax.axis_index("s")
    wid = s * 2 + c

    pltpu.sync_copy(idx_hbm.at[wid], idx_v)

    def step(j, carry):
        pltpu.async_copy(table_hbm.at[idx_v.at[j]], buf_v, gsem).wait()

        def add_row(r, inner):
            for k in range(_D // _LANES):
                sl = pl.ds(k * _LANES, _LANES)
                buf_v[r, sl] = buf_v[r, sl] + _SCALE
            return inner

        lax.fori_loop(0, _CHUNK, add_row, 0)
        pltpu.sync_copy(buf_v, out_hbm.at[wid, j])
        return carry

    lax.fori_loop(0, _NCHUNK, step, 0)


@jax.jit
def _embed(idx, table):
    f = pl.kernel(
        _body,
        out_type=jax.ShapeDtypeStruct((_NW, _NCHUNK, _CHUNK, _D), jnp.float32),
        mesh=plsc.VectorSubcoreMesh(core_axis_name="c", subcore_axis_name="s"),
        compiler_params=pltpu.CompilerParams(use_tc_tiling_on_sc=False),
        scratch_types=[
            pltpu.VMEM((_NCHUNK, _CHUNK), jnp.int32),
            pltpu.VMEM((_CHUNK, _D), jnp.float32),
            pltpu.SemaphoreType.DMA,
        ],
    )
    return f(idx, table)


def kernel(x, embedding_table):
    idx = x.astype(jnp.int32).reshape(_NW, _NCHUNK, _CHUNK)
    out = _embed(idx, embedding_table)
    return out.reshape(x.shape[0], x.shape[1], _D)
